# baseline (speedup 1.0000x reference)
import jax
import jax.numpy as jnp
from jax import lax
from jax.experimental import pallas as pl
from jax.experimental.pallas import tpu as pltpu
from jax.experimental.pallas import tpu_sc as plsc

_B = 16384
_D = 64
_LANES = 16
_NCHUNK = _D // _LANES


def _score_body(nodes_hbm, rel_hbm, s_hbm, o_hbm, r_hbm, out_hbm,
                sidx_v, oidx_v, ridx_v, pidx_v, pbuf_v, tbuf_v,
                out_v, sem):
    info = plsc.get_sparse_core_info()
    nw = info.num_cores * info.num_subcores
    bpw = _B // nw
    half = bpw // 2
    ngrp = half // _LANES
    wid = lax.axis_index("s") * info.num_cores + lax.axis_index("c")
    base = wid * bpw

    pltpu.sync_copy(s_hbm.at[pl.ds(base, bpw)], sidx_v)
    pltpu.sync_copy(o_hbm.at[pl.ds(base, bpw)], oidx_v)
    pltpu.sync_copy(r_hbm.at[pl.ds(base, bpw)], ridx_v)

    lanes = lax.iota(jnp.int32, _LANES)

    def halve(src_v, h):
        def step(k, c):
            dst = pl.ds(k * _LANES, _LANES)
            src = pl.ds(h * half + k * _LANES, _LANES)
            pidx_v[dst] = src_v[src] >> 1
            return c
        lax.fori_loop(0, ngrp, step, 0)

    for h in range(2):
        halve(ridx_v, h)
        pltpu.async_copy(rel_hbm.at[pidx_v], pbuf_v, sem).wait()

        def phase_a(g, c):
            row0 = g * _LANES
            chunk = ridx_v[pl.ds(h * half + row0, _LANES)]
            for l in range(_LANES):
                par = chunk[l] & 1
                for j in range(_NCHUNK):
                    src = pl.ds(par * _D + j * _LANES, _LANES)
                    tbuf_v[row0 + l, pl.ds(j * _LANES, _LANES)] = (
                        pbuf_v[row0 + l, src])
            return c
        lax.fori_loop(0, ngrp, phase_a, 0)

        halve(sidx_v, h)
        pltpu.async_copy(nodes_hbm.at[pidx_v], pbuf_v, sem).wait()

        def phase_b(g, c):
            row0 = g * _LANES
            chunk = sidx_v[pl.ds(h * half + row0, _LANES)]
            for l in range(_LANES):
                par = chunk[l] & 1
                for j in range(_NCHUNK):
                    dst = pl.ds(j * _LANES, _LANES)
                    src = pl.ds(par * _D + j * _LANES, _LANES)
                    tbuf_v[row0 + l, dst] = (tbuf_v[row0 + l, dst]
                                             * pbuf_v[row0 + l, src])
            return c
        lax.fori_loop(0, ngrp, phase_b, 0)

        halve(oidx_v, h)
        pltpu.async_copy(nodes_hbm.at[pidx_v], pbuf_v, sem).wait()

        def phase_c(g, c):
            row0 = g * _LANES
            chunk = oidx_v[pl.ds(h * half + row0, _LANES)]
            tot = jnp.zeros((_LANES,), jnp.float32)
            for l in range(_LANES):
                par = chunk[l] & 1
                src = pl.ds(par * _D, _LANES)
                acc = (tbuf_v[row0 + l, pl.ds(0, _LANES)]
                       * pbuf_v[row0 + l, src])
                for j in range(1, _NCHUNK):
                    dst = pl.ds(j * _LANES, _LANES)
                    src = pl.ds(par * _D + j * _LANES, _LANES)
                    acc = acc + tbuf_v[row0 + l, dst] * pbuf_v[row0 + l, src]
                tot = jnp.where(lanes == l, jnp.sum(acc), tot)
            out_v[pl.ds(h * half + row0, _LANES)] = tot
            return c
        lax.fori_loop(0, ngrp, phase_c, 0)

    pltpu.sync_copy(out_v, out_hbm.at[pl.ds(base, bpw)])


def kernel(node_embeddings, s, o, r, rel_embedding):
    info = plsc.get_sparse_core_info()
    nw = info.num_cores * info.num_subcores
    bpw = _B // nw
    mesh = plsc.VectorSubcoreMesh(core_axis_name="c", subcore_axis_name="s")
    run = pl.kernel(
        _score_body,
        out_type=jax.ShapeDtypeStruct((_B,), jnp.float32),
        mesh=mesh,
        compiler_params=pltpu.CompilerParams(needs_layout_passes=False),
        scratch_types=[
            pltpu.VMEM((bpw,), jnp.int32),
            pltpu.VMEM((bpw,), jnp.int32),
            pltpu.VMEM((bpw,), jnp.int32),
            pltpu.VMEM((bpw // 2,), jnp.int32),
            pltpu.VMEM((bpw // 2, 2 * _D), jnp.float32),
            pltpu.VMEM((bpw // 2, _D), jnp.float32),
            pltpu.VMEM((bpw,), jnp.float32),
            pltpu.SemaphoreType.DMA,
        ],
    )
    nodes2 = node_embeddings.reshape(-1, 2 * _D)
    rel2 = rel_embedding.reshape(-1, 2 * _D)
    return run(nodes2, rel2,
               s.astype(jnp.int32), o.astype(jnp.int32), r.astype(jnp.int32))


# speedup vs baseline: 1.7157x; 1.7157x over previous
import jax
import jax.numpy as jnp
from jax import lax
from jax.experimental import pallas as pl
from jax.experimental.pallas import tpu as pltpu
from jax.experimental.pallas import tpu_sc as plsc

_B = 16384
_D = 64
_LANES = 16
_NCHUNK = _D // _LANES


def _score_body(nodes_hbm, rel_hbm, s_hbm, o_hbm, r_hbm, out_hbm,
                sidx_v, oidx_v, ridx_v, pbuf_v, tbuf_v, out_v, sem):
    info = plsc.get_sparse_core_info()
    nw = info.num_cores * info.num_subcores
    bpw = _B // nw
    half = bpw // 2
    ngrp = half // _LANES
    wid = lax.axis_index("s") * info.num_cores + lax.axis_index("c")
    base = wid * bpw

    pltpu.sync_copy(s_hbm.at[pl.ds(base, bpw)], sidx_v)
    pltpu.sync_copy(o_hbm.at[pl.ds(base, bpw)], oidx_v)
    pltpu.sync_copy(r_hbm.at[pl.ds(base, bpw)], ridx_v)

    lanes = lax.iota(jnp.int32, _LANES)

    def fetch_rows(idx_v, table_hbm, h):
        def enq(g, c):
            chunk = idx_v[pl.ds(h * half + g * _LANES, _LANES)]
            for l in range(_LANES):
                pltpu.async_copy(table_hbm.at[chunk[l]],
                                 pbuf_v.at[g * _LANES + l], sem)
            return c
        lax.fori_loop(0, ngrp, enq, 0)

        def drain(i, c):
            pltpu.make_async_copy(table_hbm.at[0], pbuf_v.at[i], sem).wait()
            return c
        lax.fori_loop(0, half, drain, 0)

    for h in range(2):
        fetch_rows(ridx_v, rel_hbm, h)

        def phase_a(g, c):
            row0 = g * _LANES
            for l in range(_LANES):
                for j in range(_NCHUNK):
                    sl = pl.ds(j * _LANES, _LANES)
                    tbuf_v[row0 + l, sl] = pbuf_v[row0 + l, sl]
            return c
        lax.fori_loop(0, ngrp, phase_a, 0)

        fetch_rows(sidx_v, nodes_hbm, h)

        def phase_b(g, c):
            row0 = g * _LANES
            for l in range(_LANES):
                for j in range(_NCHUNK):
                    sl = pl.ds(j * _LANES, _LANES)
                    tbuf_v[row0 + l, sl] = (tbuf_v[row0 + l, sl]
                                            * pbuf_v[row0 + l, sl])
            return c
        lax.fori_loop(0, ngrp, phase_b, 0)

        fetch_rows(oidx_v, nodes_hbm, h)

        def phase_c(g, c):
            row0 = g * _LANES
            tot = jnp.zeros((_LANES,), jnp.float32)
            for l in range(_LANES):
                sl = pl.ds(0, _LANES)
                acc = tbuf_v[row0 + l, sl] * pbuf_v[row0 + l, sl]
                for j in range(1, _NCHUNK):
                    sl = pl.ds(j * _LANES, _LANES)
                    acc = acc + tbuf_v[row0 + l, sl] * pbuf_v[row0 + l, sl]
                tot = jnp.where(lanes == l, jnp.sum(acc), tot)
            out_v[pl.ds(h * half + row0, _LANES)] = tot
            return c
        lax.fori_loop(0, ngrp, phase_c, 0)

    pltpu.sync_copy(out_v, out_hbm.at[pl.ds(base, bpw)])


def kernel(node_embeddings, s, o, r, rel_embedding):
    info = plsc.get_sparse_core_info()
    nw = info.num_cores * info.num_subcores
    bpw = _B // nw
    mesh = plsc.VectorSubcoreMesh(core_axis_name="c", subcore_axis_name="s")
    run = pl.kernel(
        _score_body,
        out_type=jax.ShapeDtypeStruct((_B,), jnp.float32),
        mesh=mesh,
        compiler_params=pltpu.CompilerParams(needs_layout_passes=False),
        scratch_types=[
            pltpu.VMEM((bpw,), jnp.int32),
            pltpu.VMEM((bpw,), jnp.int32),
            pltpu.VMEM((bpw,), jnp.int32),
            pltpu.VMEM((bpw // 2, _D), jnp.float32),
            pltpu.VMEM((bpw // 2, _D), jnp.float32),
            pltpu.VMEM((bpw,), jnp.float32),
            pltpu.SemaphoreType.DMA,
        ],
    )
    return run(node_embeddings, rel_embedding,
               s.astype(jnp.int32), o.astype(jnp.int32), r.astype(jnp.int32))


# speedup vs baseline: 2.4487x; 1.4272x over previous
import jax
import jax.numpy as jnp
from jax import lax
from jax.experimental import pallas as pl
from jax.experimental.pallas import tpu as pltpu
from jax.experimental.pallas import tpu_sc as plsc

_B = 16384
_D = 64
_LANES = 16
_NCHUNK = _D // _LANES


def _score_body(nodes_hbm, rel_hbm, s_hbm, o_hbm, r_hbm, out_hbm,
                sidx_v, oidx_v, ridx_v, pbuf_v, tbuf_v, out_v, sem):
    info = plsc.get_sparse_core_info()
    nw = info.num_cores * info.num_subcores
    bpw = _B // nw
    half = bpw // 2
    ngrp = half // _LANES
    wid = lax.axis_index("s") * info.num_cores + lax.axis_index("c")
    base = wid * bpw

    pltpu.sync_copy(s_hbm.at[pl.ds(base, bpw)], sidx_v)
    pltpu.sync_copy(o_hbm.at[pl.ds(base, bpw)], oidx_v)
    pltpu.sync_copy(r_hbm.at[pl.ds(base, bpw)], ridx_v)

    lanes = lax.iota(jnp.int32, _LANES)

    def fetch_rows(idx_v, table_hbm, h):
        def enq(g, c):
            chunk = idx_v[pl.ds(h * half + g * _LANES, _LANES)]
            for l in range(_LANES):
                pltpu.async_copy(table_hbm.at[chunk[l]],
                                 pbuf_v.at[g * _LANES + l], sem)
            return c
        lax.fori_loop(0, ngrp, enq, 0)

        def drain(i, c):
            pltpu.make_async_copy(table_hbm.at[0], pbuf_v.at[i], sem).wait()
            return c
        lax.fori_loop(0, half, drain, 0)

    for h in range(2):
        fetch_rows(ridx_v, rel_hbm, h)

        def phase_a(g, c):
            row0 = g * _LANES
            for l in range(_LANES):
                for j in range(_NCHUNK):
                    sl = pl.ds(j * _LANES, _LANES)
                    tbuf_v[row0 + l, sl] = pbuf_v[row0 + l, sl]
            return c
        lax.fori_loop(0, ngrp, phase_a, 0)

        fetch_rows(sidx_v, nodes_hbm, h)

        def phase_b(g, c):
            row0 = g * _LANES
            for l in range(_LANES):
                for j in range(_NCHUNK):
                    sl = pl.ds(j * _LANES, _LANES)
                    tbuf_v[row0 + l, sl] = (tbuf_v[row0 + l, sl]
                                            * pbuf_v[row0 + l, sl])
            return c
        lax.fori_loop(0, ngrp, phase_b, 0)

        fetch_rows(oidx_v, nodes_hbm, h)

        def phase_c(g, c):
            row0 = g * _LANES
            tot = jnp.zeros((_LANES,), jnp.float32)
            for l in range(_LANES):
                sl = pl.ds(0, _LANES)
                acc = tbuf_v[row0 + l, sl] * pbuf_v[row0 + l, sl]
                for j in range(1, _NCHUNK):
                    sl = pl.ds(j * _LANES, _LANES)
                    acc = acc + tbuf_v[row0 + l, sl] * pbuf_v[row0 + l, sl]
                tot = jnp.where(lanes == l, jnp.sum(acc), tot)
            out_v[pl.ds(h * half + row0, _LANES)] = tot
            return c
        lax.fori_loop(0, ngrp, phase_c, 0)

    pltpu.sync_copy(out_v, out_hbm.at[pl.ds(base, bpw)])


def kernel(node_embeddings, s, o, r, rel_embedding):
    info = plsc.get_sparse_core_info()
    nw = info.num_cores * info.num_subcores
    bpw = _B // nw
    mesh = plsc.VectorSubcoreMesh(core_axis_name="c", subcore_axis_name="s")
    run = pl.kernel(
        _score_body,
        out_type=jax.ShapeDtypeStruct((_B,), jnp.float32),
        mesh=mesh,
        compiler_params=pltpu.CompilerParams(needs_layout_passes=False,
                                             use_tc_tiling_on_sc=True),
        scratch_types=[
            pltpu.VMEM((bpw,), jnp.int32),
            pltpu.VMEM((bpw,), jnp.int32),
            pltpu.VMEM((bpw,), jnp.int32),
            pltpu.VMEM((bpw // 2, _D), jnp.float32),
            pltpu.VMEM((bpw // 2, _D), jnp.float32),
            pltpu.VMEM((bpw,), jnp.float32),
            pltpu.SemaphoreType.DMA,
        ],
    )
    nodes_rm = node_embeddings.at[jnp.zeros((8,), jnp.int32)].add(
        jnp.zeros((8, _D), jnp.float32))
    return run(nodes_rm, rel_embedding,
               s.astype(jnp.int32), o.astype(jnp.int32), r.astype(jnp.int32))


# speedup vs baseline: 2.4575x; 1.0036x over previous
import jax
import jax.numpy as jnp
from jax import lax
from jax.experimental import pallas as pl
from jax.experimental.pallas import tpu as pltpu
from jax.experimental.pallas import tpu_sc as plsc

_B = 16384
_D = 64
_LANES = 16
_NCHUNK = _D // _LANES


def _score_body(nodes_hbm, rel_hbm, s_hbm, o_hbm, r_hbm, out_hbm,
                sidx_v, oidx_v, ridx_v, rbuf_v, sbuf_v, obuf_v, out_v, sem):
    info = plsc.get_sparse_core_info()
    nw = info.num_cores * info.num_subcores
    bpw = _B // nw
    half = bpw // 2
    ngrp = half // _LANES
    wid = lax.axis_index("s") * info.num_cores + lax.axis_index("c")
    base = wid * bpw

    pltpu.sync_copy(s_hbm.at[pl.ds(base, bpw)], sidx_v)
    pltpu.sync_copy(o_hbm.at[pl.ds(base, bpw)], oidx_v)
    pltpu.sync_copy(r_hbm.at[pl.ds(base, bpw)], ridx_v)

    lanes = lax.iota(jnp.int32, _LANES)

    for h in range(2):
        def enq(g, c):
            row0 = g * _LANES
            rchunk = ridx_v[pl.ds(h * half + row0, _LANES)]
            schunk = sidx_v[pl.ds(h * half + row0, _LANES)]
            ochunk = oidx_v[pl.ds(h * half + row0, _LANES)]
            for l in range(_LANES):
                pltpu.async_copy(rel_hbm.at[rchunk[l]],
                                 rbuf_v.at[row0 + l], sem)
                pltpu.async_copy(nodes_hbm.at[schunk[l]],
                                 sbuf_v.at[row0 + l], sem)
                pltpu.async_copy(nodes_hbm.at[ochunk[l]],
                                 obuf_v.at[row0 + l], sem)
            return c
        lax.fori_loop(0, ngrp, enq, 0)

        def drain(i, c):
            pltpu.make_async_copy(rel_hbm.at[0], rbuf_v.at[i], sem).wait()
            pltpu.make_async_copy(nodes_hbm.at[0], sbuf_v.at[i], sem).wait()
            pltpu.make_async_copy(nodes_hbm.at[0], obuf_v.at[i], sem).wait()
            return c
        lax.fori_loop(0, half, drain, 0)

        def compute(g, c):
            row0 = g * _LANES
            tot = jnp.zeros((_LANES,), jnp.float32)
            for l in range(_LANES):
                sl = pl.ds(0, _LANES)
                acc = (rbuf_v[row0 + l, sl] * sbuf_v[row0 + l, sl]
                       * obuf_v[row0 + l, sl])
                for j in range(1, _NCHUNK):
                    sl = pl.ds(j * _LANES, _LANES)
                    acc = acc + (rbuf_v[row0 + l, sl] * sbuf_v[row0 + l, sl]
                                 * obuf_v[row0 + l, sl])
                tot = jnp.where(lanes == l, jnp.sum(acc), tot)
            out_v[pl.ds(h * half + row0, _LANES)] = tot
            return c
        lax.fori_loop(0, ngrp, compute, 0)

    pltpu.sync_copy(out_v, out_hbm.at[pl.ds(base, bpw)])


def kernel(node_embeddings, s, o, r, rel_embedding):
    info = plsc.get_sparse_core_info()
    nw = info.num_cores * info.num_subcores
    bpw = _B // nw
    mesh = plsc.VectorSubcoreMesh(core_axis_name="c", subcore_axis_name="s")
    run = pl.kernel(
        _score_body,
        out_type=jax.ShapeDtypeStruct((_B,), jnp.float32),
        mesh=mesh,
        compiler_params=pltpu.CompilerParams(needs_layout_passes=False,
                                             use_tc_tiling_on_sc=True),
        scratch_types=[
            pltpu.VMEM((bpw,), jnp.int32),
            pltpu.VMEM((bpw,), jnp.int32),
            pltpu.VMEM((bpw,), jnp.int32),
            pltpu.VMEM((bpw // 2, _D), jnp.float32),
            pltpu.VMEM((bpw // 2, _D), jnp.float32),
            pltpu.VMEM((bpw // 2, _D), jnp.float32),
            pltpu.VMEM((bpw,), jnp.float32),
            pltpu.SemaphoreType.DMA,
        ],
    )
    nodes_rm = node_embeddings.at[jnp.zeros((8,), jnp.int32)].add(
        jnp.zeros((8, _D), jnp.float32))
    return run(nodes_rm, rel_embedding,
               s.astype(jnp.int32), o.astype(jnp.int32), r.astype(jnp.int32))


# speedup vs baseline: 2.4823x; 1.0101x over previous
import jax
import jax.numpy as jnp
from jax import lax
from jax.experimental import pallas as pl
from jax.experimental.pallas import tpu as pltpu
from jax.experimental.pallas import tpu_sc as plsc

_B = 16384
_D = 64
_LANES = 16
_NCHUNK = _D // _LANES


def _score_body(nodes_hbm, rel_hbm, s_hbm, o_hbm, r_hbm, out_hbm,
                sidx_v, oidx_v, ridx_v, rbuf_v, sbuf_v, obuf_v, out_v, sem):
    info = plsc.get_sparse_core_info()
    nw = info.num_cores * info.num_subcores
    bpw = _B // nw
    half = bpw // 2
    ngrp = half // _LANES
    wid = lax.axis_index("s") * info.num_cores + lax.axis_index("c")
    base = wid * bpw

    pltpu.sync_copy(s_hbm.at[pl.ds(base, bpw)], sidx_v)
    pltpu.sync_copy(o_hbm.at[pl.ds(base, bpw)], oidx_v)
    pltpu.sync_copy(r_hbm.at[pl.ds(base, bpw)], ridx_v)

    lanes = lax.iota(jnp.int32, _LANES)

    for h in range(2):
        def enq(g, c):
            row0 = g * _LANES
            rchunk = ridx_v[pl.ds(h * half + row0, _LANES)]
            schunk = sidx_v[pl.ds(h * half + row0, _LANES)]
            ochunk = oidx_v[pl.ds(h * half + row0, _LANES)]
            for l in range(_LANES):
                pltpu.async_copy(rel_hbm.at[rchunk[l]],
                                 rbuf_v.at[row0 + l], sem)
                pltpu.async_copy(nodes_hbm.at[schunk[l]],
                                 sbuf_v.at[row0 + l], sem)
                pltpu.async_copy(nodes_hbm.at[ochunk[l]],
                                 obuf_v.at[row0 + l], sem)
            return c
        lax.fori_loop(0, ngrp, enq, 0)

        pltpu.make_async_copy(rel_hbm.at[pl.ds(0, half)], rbuf_v, sem).wait()
        pltpu.make_async_copy(nodes_hbm.at[pl.ds(0, half)], sbuf_v,
                              sem).wait()
        pltpu.make_async_copy(nodes_hbm.at[pl.ds(0, half)], obuf_v,
                              sem).wait()

        def compute(g, c):
            row0 = g * _LANES
            tot = jnp.zeros((_LANES,), jnp.float32)
            for l in range(_LANES):
                sl = pl.ds(0, _LANES)
                acc = (rbuf_v[row0 + l, sl] * sbuf_v[row0 + l, sl]
                       * obuf_v[row0 + l, sl])
                for j in range(1, _NCHUNK):
                    sl = pl.ds(j * _LANES, _LANES)
                    acc = acc + (rbuf_v[row0 + l, sl] * sbuf_v[row0 + l, sl]
                                 * obuf_v[row0 + l, sl])
                tot = jnp.where(lanes == l, jnp.sum(acc), tot)
            out_v[pl.ds(h * half + row0, _LANES)] = tot
            return c
        lax.fori_loop(0, ngrp, compute, 0)

    pltpu.sync_copy(out_v, out_hbm.at[pl.ds(base, bpw)])


def kernel(node_embeddings, s, o, r, rel_embedding):
    info = plsc.get_sparse_core_info()
    nw = info.num_cores * info.num_subcores
    bpw = _B // nw
    mesh = plsc.VectorSubcoreMesh(core_axis_name="c", subcore_axis_name="s")
    run = pl.kernel(
        _score_body,
        out_type=jax.ShapeDtypeStruct((_B,), jnp.float32),
        mesh=mesh,
        compiler_params=pltpu.CompilerParams(needs_layout_passes=False,
                                             use_tc_tiling_on_sc=True),
        scratch_types=[
            pltpu.VMEM((bpw,), jnp.int32),
            pltpu.VMEM((bpw,), jnp.int32),
            pltpu.VMEM((bpw,), jnp.int32),
            pltpu.VMEM((bpw // 2, _D), jnp.float32),
            pltpu.VMEM((bpw // 2, _D), jnp.float32),
            pltpu.VMEM((bpw // 2, _D), jnp.float32),
            pltpu.VMEM((bpw,), jnp.float32),
            pltpu.SemaphoreType.DMA,
        ],
    )
    nodes_rm = node_embeddings.at[jnp.zeros((8,), jnp.int32)].add(
        jnp.zeros((8, _D), jnp.float32))
    return run(nodes_rm, rel_embedding,
               s.astype(jnp.int32), o.astype(jnp.int32), r.astype(jnp.int32))


# speedup vs baseline: 2.4880x; 1.0023x over previous
import jax
import jax.numpy as jnp
from jax import lax
from jax.experimental import pallas as pl
from jax.experimental.pallas import tpu as pltpu
from jax.experimental.pallas import tpu_sc as plsc

_B = 16384
_D = 64
_LANES = 16
_NCHUNK = _D // _LANES


def _score_body(nodes_hbm, rel_hbm, s_hbm, o_hbm, r_hbm, out_hbm,
                sidx_v, oidx_v, ridx_v, rbuf_v, sbuf_v, obuf_v, out_v, sem):
    info = plsc.get_sparse_core_info()
    nw = info.num_cores * info.num_subcores
    bpw = _B // nw
    half = bpw // 2
    ngrp = half // _LANES
    wid = lax.axis_index("s") * info.num_cores + lax.axis_index("c")
    base = wid * bpw

    c1 = pltpu.async_copy(s_hbm.at[pl.ds(base, bpw)], sidx_v, sem)
    c2 = pltpu.async_copy(o_hbm.at[pl.ds(base, bpw)], oidx_v, sem)
    c3 = pltpu.async_copy(r_hbm.at[pl.ds(base, bpw)], ridx_v, sem)
    c1.wait()
    c2.wait()
    c3.wait()

    lanes = lax.iota(jnp.int32, _LANES)

    for h in range(2):
        def enq(g, c):
            row0 = g * _LANES
            rchunk = ridx_v[pl.ds(h * half + row0, _LANES)]
            schunk = sidx_v[pl.ds(h * half + row0, _LANES)]
            ochunk = oidx_v[pl.ds(h * half + row0, _LANES)]
            for l in range(_LANES):
                pltpu.async_copy(rel_hbm.at[rchunk[l]],
                                 rbuf_v.at[row0 + l], sem)
                pltpu.async_copy(nodes_hbm.at[schunk[l]],
                                 sbuf_v.at[row0 + l], sem)
                pltpu.async_copy(nodes_hbm.at[ochunk[l]],
                                 obuf_v.at[row0 + l], sem)
            return c
        lax.fori_loop(0, ngrp, enq, 0)

        pltpu.make_async_copy(rel_hbm.at[pl.ds(0, half)], rbuf_v, sem).wait()
        pltpu.make_async_copy(nodes_hbm.at[pl.ds(0, half)], sbuf_v,
                              sem).wait()
        pltpu.make_async_copy(nodes_hbm.at[pl.ds(0, half)], obuf_v,
                              sem).wait()

        def compute(g, c):
            row0 = g * _LANES
            tot = jnp.zeros((_LANES,), jnp.float32)
            for l in range(_LANES):
                sl = pl.ds(0, _LANES)
                acc = (rbuf_v[row0 + l, sl] * sbuf_v[row0 + l, sl]
                       * obuf_v[row0 + l, sl])
                for j in range(1, _NCHUNK):
                    sl = pl.ds(j * _LANES, _LANES)
                    acc = acc + (rbuf_v[row0 + l, sl] * sbuf_v[row0 + l, sl]
                                 * obuf_v[row0 + l, sl])
                tot = jnp.where(lanes == l, jnp.sum(acc), tot)
            out_v[pl.ds(h * half + row0, _LANES)] = tot
            return c
        lax.fori_loop(0, ngrp, compute, 0)

    pltpu.sync_copy(out_v, out_hbm.at[pl.ds(base, bpw)])


def kernel(node_embeddings, s, o, r, rel_embedding):
    info = plsc.get_sparse_core_info()
    nw = info.num_cores * info.num_subcores
    bpw = _B // nw
    mesh = plsc.VectorSubcoreMesh(core_axis_name="c", subcore_axis_name="s")
    run = pl.kernel(
        _score_body,
        out_type=jax.ShapeDtypeStruct((_B,), jnp.float32),
        mesh=mesh,
        compiler_params=pltpu.CompilerParams(needs_layout_passes=False,
                                             use_tc_tiling_on_sc=True),
        scratch_types=[
            pltpu.VMEM((bpw,), jnp.int32),
            pltpu.VMEM((bpw,), jnp.int32),
            pltpu.VMEM((bpw,), jnp.int32),
            pltpu.VMEM((bpw // 2, _D), jnp.float32),
            pltpu.VMEM((bpw // 2, _D), jnp.float32),
            pltpu.VMEM((bpw // 2, _D), jnp.float32),
            pltpu.VMEM((bpw,), jnp.float32),
            pltpu.SemaphoreType.DMA,
        ],
    )
    nodes_rm = node_embeddings.at[jnp.zeros((8,), jnp.int32)].add(
        jnp.zeros((8, _D), jnp.float32))
    return run(nodes_rm, rel_embedding,
               s.astype(jnp.int32), o.astype(jnp.int32), r.astype(jnp.int32))
